# Initial kernel scaffold; baseline (speedup 1.0000x reference)
import functools

import jax
import jax.numpy as jnp
from jax import lax
from jax.experimental import pallas as pl
from jax.experimental.pallas import tpu as pltpu
from jax.experimental.pallas import tpu_sc as plsc

B = 16384
S = 200
E = 64
NC = 2
NS = 16
NW = NC * NS
BPW = B // NW
H = S // 2
CH = 8
HPC = CH * 2
NCHUNK = BPW // CH
NLANE = 4

_mesh = plsc.VectorSubcoreMesh(core_axis_name="c", subcore_axis_name="s")


@functools.partial(
    pl.kernel,
    out_type=jax.ShapeDtypeStruct((B, E), jnp.float32),
    mesh=_mesh,
    scratch_types=[
        pltpu.VMEM((HPC, H), jnp.int32),
        pltpu.VMEM((2, H, E), jnp.float32),
        pltpu.VMEM((CH, E), jnp.float32),
        pltpu.SemaphoreType.DMA,
        pltpu.SemaphoreType.DMA,
    ],
)
def _pooled_lookup(x_hbm, table_hbm, out_hbm, idx_v, rows_v, out_v, sem0, sem1):
    wid = lax.axis_index("s") * NC + lax.axis_index("c")
    sems = (sem0, sem1)

    def chunk_body(ci, carry):
        hbase = (wid * BPW + ci * CH) * 2
        pltpu.sync_copy(x_hbm.at[pl.ds(hbase, HPC)], idx_v)
        pltpu.make_async_copy(
            table_hbm.at[idx_v.at[0]], rows_v.at[0], sems[0]
        ).start()
        for s in range(CH):
            acc = tuple(jnp.zeros((16,), jnp.float32) for _ in range(NLANE))
            for hh in range(2):
                h = 2 * s + hh
                buf = h % 2
                pltpu.make_async_copy(
                    table_hbm.at[idx_v.at[h]], rows_v.at[buf], sems[buf]
                ).wait()
                if h + 1 < HPC:
                    nbuf = (h + 1) % 2
                    pltpu.make_async_copy(
                        table_hbm.at[idx_v.at[h + 1]], rows_v.at[nbuf], sems[nbuf]
                    ).start()

                def red(i, a, _buf=buf):
                    return tuple(
                        a[c] + rows_v[_buf, i, pl.ds(c * 16, 16)]
                        for c in range(NLANE)
                    )

                acc = lax.fori_loop(0, H, red, acc)
            for c in range(NLANE):
                out_v[s, pl.ds(c * 16, 16)] = acc[c] * (1.0 / S)
        pltpu.sync_copy(out_v, out_hbm.at[pl.ds(wid * BPW + ci * CH, CH)])
        return carry

    lax.fori_loop(0, NCHUNK, chunk_body, 0)


def kernel(x, x_len, table):
    del x_len
    x_halves = x.reshape(B * 2, H)
    return _pooled_lookup(x_halves, table)


# speedup vs baseline: 1.9798x; 1.9798x over previous
import functools

import jax
import jax.numpy as jnp
from jax import lax
from jax.experimental import pallas as pl
from jax.experimental.pallas import tpu as pltpu
from jax.experimental.pallas import tpu_sc as plsc

B = 16384
S = 200
E = 64
NC = 2
NS = 16
NW = NC * NS
BPW = B // NW
H = S // 2
CH = 8
HPC = CH * 2
NCHUNK = BPW // CH
NLANE = 4

_mesh = plsc.VectorSubcoreMesh(core_axis_name="c", subcore_axis_name="s")


@functools.partial(
    pl.kernel,
    out_type=jax.ShapeDtypeStruct((B, E), jnp.float32),
    mesh=_mesh,
    compiler_params=pltpu.CompilerParams(use_tc_tiling_on_sc=False),
    scratch_types=[
        pltpu.VMEM((HPC, H), jnp.int32),
        pltpu.VMEM((2, H, E), jnp.float32),
        pltpu.VMEM((CH, E), jnp.float32),
        pltpu.SemaphoreType.DMA,
        pltpu.SemaphoreType.DMA,
    ],
)
def _pooled_lookup(x_hbm, table_hbm, out_hbm, idx_v, rows_v, out_v, sem0, sem1):
    wid = lax.axis_index("s") * NC + lax.axis_index("c")
    sems = (sem0, sem1)

    def chunk_body(ci, carry):
        hbase = (wid * BPW + ci * CH) * 2
        pltpu.sync_copy(x_hbm.at[pl.ds(hbase, HPC)], idx_v)
        pltpu.make_async_copy(
            table_hbm.at[idx_v.at[0]], rows_v.at[0], sems[0]
        ).start()
        for s in range(CH):
            acc = tuple(jnp.zeros((16,), jnp.float32) for _ in range(NLANE))
            for hh in range(2):
                h = 2 * s + hh
                buf = h % 2
                pltpu.make_async_copy(
                    table_hbm.at[idx_v.at[h]], rows_v.at[buf], sems[buf]
                ).wait()
                if h + 1 < HPC:
                    nbuf = (h + 1) % 2
                    pltpu.make_async_copy(
                        table_hbm.at[idx_v.at[h + 1]], rows_v.at[nbuf], sems[nbuf]
                    ).start()

                def red(i, a, _buf=buf):
                    return tuple(
                        a[c] + rows_v[_buf, i, pl.ds(c * 16, 16)]
                        for c in range(NLANE)
                    )

                acc = lax.fori_loop(0, H, red, acc)
            for c in range(NLANE):
                out_v[s, pl.ds(c * 16, 16)] = acc[c] * (1.0 / S)
        pltpu.sync_copy(out_v, out_hbm.at[pl.ds(wid * BPW + ci * CH, CH)])
        return carry

    lax.fori_loop(0, NCHUNK, chunk_body, 0)


def kernel(x, x_len, table):
    del x_len
    x_halves = x.reshape(B * 2, H)
    return _pooled_lookup(x_halves, table)


# speedup vs baseline: 3.4217x; 1.7283x over previous
import functools

import jax
import jax.numpy as jnp
from jax import lax
from jax.experimental import pallas as pl
from jax.experimental.pallas import tpu as pltpu
from jax.experimental.pallas import tpu_sc as plsc

B = 16384
S = 200
E = 64
NC = 2
NS = 16
NW = NC * NS
BPW = B // NW
H = S // 2
CH = 8
HPC = CH * 2
NCHUNK = BPW // CH
NB = NCHUNK // 2
RING = 8
NLANE = 4

_mesh = plsc.VectorSubcoreMesh(core_axis_name="c", subcore_axis_name="s")


@functools.partial(
    pl.kernel,
    out_type=jax.ShapeDtypeStruct((B, E), jnp.float32),
    mesh=_mesh,
    compiler_params=pltpu.CompilerParams(use_tc_tiling_on_sc=False),
    scratch_types=[
        pltpu.VMEM((2, HPC, H), jnp.int32),
        pltpu.VMEM((RING, H, E), jnp.float32),
        pltpu.VMEM((2, CH, E), jnp.float32),
        pltpu.SemaphoreType.DMA((RING,)),
        pltpu.SemaphoreType.DMA((2,)),
        pltpu.SemaphoreType.DMA((2,)),
    ],
)
def _pooled_lookup(x_hbm, table_hbm, out_hbm, idx_v, rows_v, out_v,
                   gsem, isem, osem):
    wid = lax.axis_index("s") * NC + lax.axis_index("c")
    wbase_s = wid * BPW
    wbase_h = wbase_s * 2

    def idx_copy(chunk, buf):
        return pltpu.make_async_copy(
            x_hbm.at[pl.ds(wbase_h + chunk * HPC, HPC)],
            idx_v.at[buf], isem.at[buf])

    def gather(ibuf, irow, slot):
        return pltpu.make_async_copy(
            table_hbm.at[idx_v.at[ibuf, irow]], rows_v.at[slot],
            gsem.at[slot])

    def out_copy(chunk, buf):
        return pltpu.make_async_copy(
            out_v.at[buf], out_hbm.at[pl.ds(wbase_s + chunk * CH, CH)],
            osem.at[buf])

    idx_copy(0, 0).start()
    idx_copy(0, 0).wait()
    for k in range(RING):
        gather(0, k, k).start()

    def body(ci2, carry):
        not_last = ci2 < NB - 1
        acc = tuple(jnp.zeros((16,), jnp.float32) for _ in range(2 * NLANE))
        for hp in range(2 * HPC):
            slot = hp % RING
            pc = hp // HPC

            if hp == 0:
                idx_copy(2 * ci2 + 1, 1).start()

                @pl.when(ci2 > 0)
                def _():
                    out_copy(0, 0).wait()
            if hp == RING:
                idx_copy(0, 1).wait()
            if hp == HPC:
                @pl.when(ci2 > 0)
                def _():
                    out_copy(0, 1).wait()

                @pl.when(not_last)
                def _():
                    idx_copy(2 * ci2 + 2, 0).start()
            if hp == 2 * HPC - RING:
                @pl.when(not_last)
                def _():
                    idx_copy(0, 0).wait()

            gather(pc, hp % HPC, slot).wait()

            def red(i, a, _slot=slot):
                a = list(a)
                r = i * 4
                for rr in range(4):
                    p = rr % 2
                    for c in range(NLANE):
                        a[c * 2 + p] = a[c * 2 + p] + rows_v[
                            _slot, r + rr, pl.ds(c * 16, 16)]
                return tuple(a)

            acc = lax.fori_loop(0, H // 4, red, acc)

            h2 = hp + RING
            if h2 < 2 * HPC:
                gather(h2 // HPC, h2 % HPC, slot).start()
            else:
                @pl.when(not_last)
                def _():
                    gather(0, h2 - 2 * HPC, slot).start()

            if hp % 2 == 1:
                sp = (hp // 2) % CH
                for c in range(NLANE):
                    out_v[pc, sp, pl.ds(c * 16, 16)] = (
                        acc[c * 2] + acc[c * 2 + 1]) * (1.0 / S)
                acc = tuple(jnp.zeros((16,), jnp.float32)
                            for _ in range(2 * NLANE))
            if hp == HPC - 1:
                out_copy(2 * ci2, 0).start()
            if hp == 2 * HPC - 1:
                out_copy(2 * ci2 + 1, 1).start()
        return carry

    lax.fori_loop(0, NB, body, 0)
    out_copy(0, 0).wait()
    out_copy(0, 1).wait()


def kernel(x, x_len, table):
    del x_len
    x_halves = x.reshape(B * 2, H)
    return _pooled_lookup(x_halves, table)
